# Initial kernel scaffold; baseline (speedup 1.0000x reference)
import jax
import jax.numpy as jnp
from jax.experimental import pallas as pl


def kernel(z_e, codebook):
    raise NotImplementedError("write your pallas kernel here")


# speedup vs baseline: 1.8865x; 1.8865x over previous
import jax
import jax.numpy as jnp
from jax.experimental import pallas as pl

LATENT = 64
CODES = 1024
PIX = 1024


def _vq_block(z_ref, cb_ref, zq_ref, idx_ref):
    z = z_ref[0]
    cb = cb_ref[...]
    cb_sq = jnp.sum(cb * cb, axis=1, keepdims=True)
    scores = cb_sq - 2.0 * jax.lax.dot_general(
        cb, z, (((1,), (0,)), ((), ())), preferred_element_type=jnp.float32
    )
    min_val = jnp.min(scores, axis=0, keepdims=True)
    code_iota = jax.lax.broadcasted_iota(jnp.int32, (CODES, PIX), 0)
    idx = jnp.min(
        jnp.where(scores == min_val, code_iota, CODES), axis=0, keepdims=True
    )
    idx_ref[0] = idx
    onehot = (code_iota == idx).astype(jnp.float32)
    zq_ref[0] = jax.lax.dot_general(
        onehot, cb, (((0,), (0,)), ((), ())), preferred_element_type=jnp.float32
    )


def kernel(z_e, codebook):
    B, C, H, W = z_e.shape
    n_pix = B * H * W
    nb = n_pix // PIX
    z3 = z_e.reshape(B, C, H * W)
    zq, idx = pl.pallas_call(
        _vq_block,
        grid=(nb,),
        in_specs=[
            pl.BlockSpec((1, C, PIX), lambda i: (i, 0, 0)),
            pl.BlockSpec((CODES, LATENT), lambda i: (0, 0)),
        ],
        out_specs=[
            pl.BlockSpec((1, PIX, LATENT), lambda i: (i, 0, 0)),
            pl.BlockSpec((1, 1, PIX), lambda i: (i, 0, 0)),
        ],
        out_shape=[
            jax.ShapeDtypeStruct((nb, PIX, LATENT), jnp.float32),
            jax.ShapeDtypeStruct((nb, 1, PIX), jnp.int32),
        ],
    )(z3, codebook)
    return zq.reshape(n_pix, LATENT), idx.reshape(n_pix)
